# baseline (speedup 1.0000x reference)
import functools
import math

import jax
import jax.numpy as jnp
from jax import lax
from jax.experimental import pallas as pl
from jax.experimental.pallas import tpu as pltpu
from jax.experimental.pallas import tpu_sc as plsc

_T = 512
_P = 64
_KPT = 32
_N = _T * _P
_EPS = 1e-5
_AREA = 1.0

_SDAYS = 160
_TDAYS = _T - _SDAYS

_NW = 32
_DPW = _SDAYS // _NW
_HD = _DPW + _KPT
_HE = _HD * _P
_OE = _DPW * _P
_NQ = _P // 16


def _sc_kers_body(xp_hbm, yp_hbm, ws_hbm, out_hbm, xv, yv, wv, ov):
    cid = lax.axis_index("c")
    sid = lax.axis_index("s")
    w = cid * 16 + sid
    base = w * _OE
    pltpu.sync_copy(xp_hbm.at[pl.ds(base, _HE)], xv)
    pltpu.sync_copy(yp_hbm.at[pl.ds(base, _HE)], yv)
    pltpu.sync_copy(ws_hbm, wv)
    d0base = w * _DPW

    _NC = _KPT * _P // 16
    _G = 8

    def day_body(dd, carry):
        d0 = d0base + dd
        cmin = jnp.maximum(jnp.int32(0), jnp.int32(_NC) - 4 * d0)
        cb = (_KPT + dd) * _P
        hb0 = dd * _P
        z = jnp.zeros((16,), jnp.float32)
        for q in range(_NQ):
            cxq = xv[pl.ds(cb + 16 * q, 16)]
            cyq = yv[pl.ds(cb + 16 * q, 16)]
            for half in range(16 // _G):
                xsp = [jnp.full((16,), cxq[half * _G + j], jnp.float32)
                       for j in range(_G)]
                ysp = [jnp.full((16,), cyq[half * _G + j], jnp.float32)
                       for j in range(_G)]

                def c_body(c, accs):
                    a = list(accs)
                    xc = xv[pl.ds(hb0 + c * 16, 16)]
                    yc = yv[pl.ds(hb0 + c * 16, 16)]
                    lw = wv[pl.ds(c * 16, 16)]
                    for j in range(_G):
                        dx = xsp[j] - xc
                        dy = ysp[j] - yc
                        t = (lw - dx * dx) - dy * dy
                        a[j] = a[j] + jnp.exp(t)
                    return tuple(a)

                accs = lax.fori_loop(cmin, _NC, c_body, (z,) * _G)
                ob = (dd * _P + q * 16 + half * _G) * 16
                for j in range(_G):
                    ov[pl.ds(ob + j * 16, 16)] = accs[j]
        return carry

    lax.fori_loop(0, _DPW, day_body, 0)
    pltpu.sync_copy(ov, out_hbm.at[pl.ds(base * 16, _OE * 16)])


_sc_kers = functools.partial(
    pl.kernel,
    out_type=jax.ShapeDtypeStruct((_SDAYS * _P * 16,), jnp.float32),
    mesh=plsc.VectorSubcoreMesh(
        core_axis_name="c", subcore_axis_name="s", num_cores=2, num_subcores=16
    ),
    scratch_types=[
        pltpu.VMEM((_HE,), jnp.float32),
        pltpu.VMEM((_HE,), jnp.float32),
        pltpu.VMEM((_KPT * _P,), jnp.float32),
        pltpu.VMEM((_OE * 16,), jnp.float32),
    ],
)(_sc_kers_body)


_WE = _KPT * _P
_DB = 4
_SLAB = _WE + _DB * _P


def _tc_pair_body(xf_ref, yf_ref, lw_ref, out_ref):
    d = _SDAYS + _DB * pl.program_id(0)
    hxs = xf_ref[0, pl.ds((d - _KPT) * _P, _SLAB)]
    hys = yf_ref[0, pl.ds((d - _KPT) * _P, _SLAB)]
    cc_x = xf_ref[0, pl.ds(d * _P, _DB * _P)]
    cc_y = yf_ref[0, pl.ds(d * _P, _DB * _P)]
    lw = lw_ref[0, :]
    es = []
    for h in range(_DB):
        hx = lax.slice(hxs, (h * _P,), (h * _P + _WE,))
        hy = lax.slice(hys, (h * _P,), (h * _P + _WE,))
        cx = lax.slice(cc_x, (h * _P,), ((h + 1) * _P,))
        cy = lax.slice(cc_y, (h * _P,), ((h + 1) * _P,))
        dx = cx[:, None] - hx[None, :]
        dy = cy[:, None] - hy[None, :]
        t = (lw[None, :] - dx * dx) - dy * dy
        es.append(jnp.exp(t).astype(jnp.bfloat16))
    e_both = jnp.concatenate(es, axis=0)
    ones_col = jnp.ones((_WE, 1), jnp.bfloat16)
    ks = lax.dot_general(e_both, ones_col, (((1,), (0,)), ((), ())),
                         preferred_element_type=jnp.float32)
    out_ref[...] = ks.reshape(1, _DB * _P, 1)


_tc_pair = pl.pallas_call(
    _tc_pair_body,
    grid=(_TDAYS // _DB,),
    out_shape=jax.ShapeDtypeStruct((_TDAYS // _DB, _DB * _P, 1), jnp.float32),
    in_specs=[
        pl.BlockSpec(memory_space=pltpu.VMEM),
        pl.BlockSpec(memory_space=pltpu.VMEM),
        pl.BlockSpec(memory_space=pltpu.VMEM),
    ],
    out_specs=pl.BlockSpec((1, _DB * _P, 1), lambda g: (g, 0, 0)),
)


def _tc_reduce_body(psc_ref, ktc_ref, day_ref, scal_ref, ll_ref, l1_ref, l2_ref):
    lam0 = scal_ref[0]
    bb = scal_ref[1]
    aa = scal_ref[2]
    part = psc_ref[...]
    kers_sc = part.reshape(_SDAYS * _P // 8, 8, 16).sum(axis=-1)
    lams1 = jnp.sum(jnp.log(kers_sc + (lam0 + _EPS)))
    lams1 = lams1 + jnp.sum(jnp.log(ktc_ref[...] + (lam0 + _EPS)))
    day = day_ref[...]
    rem = jnp.clip(jnp.float32(_T) - day, 0.0, jnp.float32(_KPT))
    edo = jnp.sum(aa * (1.0 - jnp.exp(-bb * rem)))
    lams2 = lam0 * (_AREA * _T) + edo
    l1_ref[0, 0] = lams1
    l2_ref[0, 0] = lams2
    ll_ref[0, 0] = lams1 - lams2


_tc_reduce = pl.pallas_call(
    _tc_reduce_body,
    out_shape=[
        jax.ShapeDtypeStruct((1, 1), jnp.float32),
        jax.ShapeDtypeStruct((1, 1), jnp.float32),
        jax.ShapeDtypeStruct((1, 1), jnp.float32),
    ],
    in_specs=[
        pl.BlockSpec(memory_space=pltpu.VMEM),
        pl.BlockSpec(memory_space=pltpu.VMEM),
        pl.BlockSpec(memory_space=pltpu.VMEM),
        pl.BlockSpec(memory_space=pltpu.SMEM),
    ],
    out_specs=[
        pl.BlockSpec(memory_space=pltpu.SMEM),
        pl.BlockSpec(memory_space=pltpu.SMEM),
        pl.BlockSpec(memory_space=pltpu.SMEM),
    ],
)


def kernel(obs, Lambda0, C, beta, sigma):
    lam0 = Lambda0[0]
    c = C[0]
    b = beta[0]
    sg = sigma[0]

    day = obs[:, 0]
    scale = 1.0 / (jnp.sqrt(2.0) * sg)
    xs = obs[:, 1] * scale
    ys = obs[:, 2] * scale
    zpad = jnp.zeros((_KPT * _P,), jnp.float32)
    xp = jnp.concatenate([zpad, xs])
    yp = jnp.concatenate([zpad, ys])

    norm = 1.0 / (2.0 * math.pi * sg * sg)
    kc = (_KPT - jnp.arange(_KPT * 4, dtype=jnp.float32) // 4.0).repeat(16)
    lw = jnp.log(c * b * norm) - b * kc
    wsplat = lw.astype(jnp.float32)

    psc = _sc_kers(xp, yp, wsplat)
    ktc = _tc_pair(xs.reshape(1, _N), ys.reshape(1, _N),
                   wsplat.reshape(1, _KPT * _P))

    eb = jnp.exp(-b)
    aa = c * b * eb / (1.0 - eb)
    scal = jnp.stack([lam0, b, aa])
    ll, l1, l2 = _tc_reduce(
        psc.reshape(_SDAYS * _P * 16 // 128, 128),
        ktc.reshape(_TDAYS * _P // 128, 128),
        day.reshape(_N // 128, 128),
        scal,
    )
    return ll[0, 0], l1[0, 0], l2[0, 0]


# speedup vs baseline: 1.0014x; 1.0014x over previous
import functools
import math

import jax
import jax.numpy as jnp
from jax import lax
from jax.experimental import pallas as pl
from jax.experimental.pallas import tpu as pltpu
from jax.experimental.pallas import tpu_sc as plsc

_T = 512
_P = 64
_KPT = 32
_N = _T * _P
_EPS = 1e-5
_AREA = 1.0

_SDAYS = 192
_TDAYS = _T - _SDAYS

_NW = 32
_DPW = _SDAYS // _NW
_HD = _DPW + _KPT
_HE = _HD * _P
_OE = _DPW * _P
_NQ = _P // 16


def _sc_kers_body(xp_hbm, yp_hbm, ws_hbm, out_hbm, xv, yv, wv, ov):
    cid = lax.axis_index("c")
    sid = lax.axis_index("s")
    w = cid * 16 + sid
    base = w * _OE
    pltpu.sync_copy(xp_hbm.at[pl.ds(base, _HE)], xv)
    pltpu.sync_copy(yp_hbm.at[pl.ds(base, _HE)], yv)
    pltpu.sync_copy(ws_hbm, wv)
    d0base = w * _DPW

    _NC = _KPT * _P // 16
    _G = 8
    lanes = lax.iota(jnp.int32, 16)

    def day_body(dd, carry):
        d0 = d0base + dd
        cmin = jnp.maximum(jnp.int32(0), jnp.int32(_NC) - 4 * d0)
        cb = (_KPT + dd) * _P
        hb0 = dd * _P
        z = jnp.zeros((16,), jnp.float32)
        for q in range(_NQ):
            cxq = xv[pl.ds(cb + 16 * q, 16)]
            cyq = yv[pl.ds(cb + 16 * q, 16)]
            outq = z
            for half in range(16 // _G):
                xsp = [jnp.full((16,), cxq[half * _G + j], jnp.float32)
                       for j in range(_G)]
                ysp = [jnp.full((16,), cyq[half * _G + j], jnp.float32)
                       for j in range(_G)]

                def c_body(c, accs):
                    a = list(accs)
                    xc = xv[pl.ds(hb0 + c * 16, 16)]
                    yc = yv[pl.ds(hb0 + c * 16, 16)]
                    lw = wv[pl.ds(c * 16, 16)]
                    for j in range(_G):
                        dx = xsp[j] - xc
                        dy = ysp[j] - yc
                        t = (lw - dx * dx) - dy * dy
                        a[j] = a[j] + jnp.exp(t)
                    return tuple(a)

                accs = lax.fori_loop(cmin, _NC, c_body, (z,) * _G)
                for j in range(_G):
                    acc = accs[j]
                    bs = [jnp.full((16,), acc[l], jnp.float32)
                          for l in range(16)]
                    while len(bs) > 1:
                        bs = [bs[i] + bs[i + 1] for i in range(0, len(bs), 2)]
                    outq = jnp.where(lanes == half * _G + j, bs[0], outq)
            ov[pl.ds(dd * _P + q * 16, 16)] = outq
        return carry

    lax.fori_loop(0, _DPW, day_body, 0)
    pltpu.sync_copy(ov, out_hbm.at[pl.ds(base, _OE)])


_sc_kers = functools.partial(
    pl.kernel,
    out_type=jax.ShapeDtypeStruct((_SDAYS * _P,), jnp.float32),
    mesh=plsc.VectorSubcoreMesh(
        core_axis_name="c", subcore_axis_name="s", num_cores=2, num_subcores=16
    ),
    scratch_types=[
        pltpu.VMEM((_HE,), jnp.float32),
        pltpu.VMEM((_HE,), jnp.float32),
        pltpu.VMEM((_KPT * _P,), jnp.float32),
        pltpu.VMEM((_OE,), jnp.float32),
    ],
)(_sc_kers_body)


_WE = _KPT * _P
_DB = 4
_SLAB = _WE + _DB * _P


def _tc_pair_body(xf_ref, yf_ref, lw_ref, out_ref):
    d = _SDAYS + _DB * pl.program_id(0)
    hxs = xf_ref[0, pl.ds((d - _KPT) * _P, _SLAB)]
    hys = yf_ref[0, pl.ds((d - _KPT) * _P, _SLAB)]
    cc_x = xf_ref[0, pl.ds(d * _P, _DB * _P)]
    cc_y = yf_ref[0, pl.ds(d * _P, _DB * _P)]
    lw = lw_ref[0, :]
    es = []
    for h in range(_DB):
        hx = lax.slice(hxs, (h * _P,), (h * _P + _WE,))
        hy = lax.slice(hys, (h * _P,), (h * _P + _WE,))
        cx = lax.slice(cc_x, (h * _P,), ((h + 1) * _P,))
        cy = lax.slice(cc_y, (h * _P,), ((h + 1) * _P,))
        dx = cx[:, None] - hx[None, :]
        dy = cy[:, None] - hy[None, :]
        t = (lw[None, :] - dx * dx) - dy * dy
        es.append(jnp.exp(t).astype(jnp.bfloat16))
    e_both = jnp.concatenate(es, axis=0)
    ones_col = jnp.ones((_WE, 1), jnp.bfloat16)
    ks = lax.dot_general(e_both, ones_col, (((1,), (0,)), ((), ())),
                         preferred_element_type=jnp.float32)
    out_ref[...] = ks.reshape(1, _DB * _P, 1)


_tc_pair = pl.pallas_call(
    _tc_pair_body,
    grid=(_TDAYS // _DB,),
    out_shape=jax.ShapeDtypeStruct((_TDAYS // _DB, _DB * _P, 1), jnp.float32),
    in_specs=[
        pl.BlockSpec(memory_space=pltpu.VMEM),
        pl.BlockSpec(memory_space=pltpu.VMEM),
        pl.BlockSpec(memory_space=pltpu.VMEM),
    ],
    out_specs=pl.BlockSpec((1, _DB * _P, 1), lambda g: (g, 0, 0)),
)


def _tc_reduce_body(psc_ref, ktc_ref, day_ref, scal_ref, ll_ref, l1_ref, l2_ref):
    lam0 = scal_ref[0]
    bb = scal_ref[1]
    aa = scal_ref[2]
    lams1 = jnp.sum(jnp.log(psc_ref[...] + (lam0 + _EPS)))
    lams1 = lams1 + jnp.sum(jnp.log(ktc_ref[...] + (lam0 + _EPS)))
    day = day_ref[...]
    rem = jnp.clip(jnp.float32(_T) - day, 0.0, jnp.float32(_KPT))
    edo = jnp.sum(aa * (1.0 - jnp.exp(-bb * rem)))
    lams2 = lam0 * (_AREA * _T) + edo
    l1_ref[0, 0] = lams1
    l2_ref[0, 0] = lams2
    ll_ref[0, 0] = lams1 - lams2


_tc_reduce = pl.pallas_call(
    _tc_reduce_body,
    out_shape=[
        jax.ShapeDtypeStruct((1, 1), jnp.float32),
        jax.ShapeDtypeStruct((1, 1), jnp.float32),
        jax.ShapeDtypeStruct((1, 1), jnp.float32),
    ],
    in_specs=[
        pl.BlockSpec(memory_space=pltpu.VMEM),
        pl.BlockSpec(memory_space=pltpu.VMEM),
        pl.BlockSpec(memory_space=pltpu.VMEM),
        pl.BlockSpec(memory_space=pltpu.SMEM),
    ],
    out_specs=[
        pl.BlockSpec(memory_space=pltpu.SMEM),
        pl.BlockSpec(memory_space=pltpu.SMEM),
        pl.BlockSpec(memory_space=pltpu.SMEM),
    ],
)


def kernel(obs, Lambda0, C, beta, sigma):
    lam0 = Lambda0[0]
    c = C[0]
    b = beta[0]
    sg = sigma[0]

    day = obs[:, 0]
    scale = 1.0 / (jnp.sqrt(2.0) * sg)
    xs = obs[:, 1] * scale
    ys = obs[:, 2] * scale
    zpad = jnp.zeros((_KPT * _P,), jnp.float32)
    xp = jnp.concatenate([zpad, xs])
    yp = jnp.concatenate([zpad, ys])

    norm = 1.0 / (2.0 * math.pi * sg * sg)
    kc = (_KPT - jnp.arange(_KPT * 4, dtype=jnp.float32) // 4.0).repeat(16)
    lw = jnp.log(c * b * norm) - b * kc
    wsplat = lw.astype(jnp.float32)

    psc = _sc_kers(xp, yp, wsplat)
    ktc = _tc_pair(xs.reshape(1, _N), ys.reshape(1, _N),
                   wsplat.reshape(1, _KPT * _P))

    eb = jnp.exp(-b)
    aa = c * b * eb / (1.0 - eb)
    scal = jnp.stack([lam0, b, aa])
    ll, l1, l2 = _tc_reduce(
        psc.reshape(_SDAYS * _P // 128, 128),
        ktc.reshape(_TDAYS * _P // 128, 128),
        day.reshape(_N // 128, 128),
        scal,
    )
    return ll[0, 0], l1[0, 0], l2[0, 0]
